# baseline (speedup 1.0000x reference)
import functools

import jax
import jax.numpy as jnp
from jax import lax
from jax.experimental import pallas as pl
from jax.experimental.pallas import tpu as pltpu
from jax.experimental.pallas import tpu_sc as plsc

BATCH = 2048
NB1 = 32
FEAT = 256
HID = 256
OUT = 128
HEADS = 2
BB = 8
ROWS = BB * NB1
GW = 128


def _leaky(x):
    return jnp.where(x >= 0, x, 0.2 * x)


def _gather(table, idx):
    n = idx.shape[1]
    mesh = plsc.VectorSubcoreMesh(core_axis_name="core", subcore_axis_name="subcore")

    @functools.partial(
        pl.kernel,
        out_type=jax.ShapeDtypeStruct((n, FEAT), jnp.float32),
        mesh=mesh,
    )
    def gk(table_hbm, idx_hbm, out_hbm):
        def body(i_vmem, o_vmem):
            pltpu.sync_copy(table_hbm.at[i_vmem.at[0]], o_vmem)

        pltpu.emit_pipeline(
            body,
            grid=(n // GW,),
            in_specs=[pl.BlockSpec((1, GW), index_map=lambda i: (0, i))],
            out_specs=[pl.BlockSpec((GW, FEAT), index_map=lambda i: (i, 0))],
            core_axis_name=("core", "subcore"),
            dimension_semantics=(pltpu.PARALLEL,),
        )(idx_hbm, out_hbm)

    return gk(table, idx)


def _att_body(seq_ref, W1_ref, Wf1_ref, fb1_ref, fbc1_ref,
              W2_ref, Wf2_ref, fb2_ref, fbc2_ref, s2bias_ref, bmean_ref, out_ref):
    X = seq_ref[...].astype(jnp.bfloat16)
    rid = lax.broadcasted_iota(jnp.int32, (ROWS, ROWS), 0) // NB1
    cid = lax.broadcasted_iota(jnp.int32, (ROWS, ROWS), 1) // NB1
    maskb = (rid == cid).astype(jnp.bfloat16)

    Wf1 = Wf1_ref[...].astype(jnp.bfloat16)
    F1 = (jnp.dot(X, Wf1, preferred_element_type=jnp.float32)
          + fb1_ref[0:1, :]).astype(jnp.bfloat16)
    G1 = (lax.dot_general(Wf1, X, (((0,), (1,)), ((), ())),
                          preferred_element_type=jnp.float32)
          + fbc1_ref[...]).astype(jnp.bfloat16)

    Sall = jnp.dot(X, W1_ref[...].astype(jnp.bfloat16),
                   preferred_element_type=jnp.float32).astype(jnp.bfloat16)
    vals = []
    for h in range(HEADS):
        Z = F1[:, 2 * h:2 * h + 1] + G1[2 * h + 1:2 * h + 2, :]
        Z = jnp.maximum(Z, jnp.bfloat16(0.2) * Z)
        E = jnp.exp(Z) * maskb
        denom = jnp.sum(E.astype(jnp.float32), axis=-1, keepdims=True)
        A = E * (1.0 / denom).astype(jnp.bfloat16)
        vals.append(jnp.dot(A, Sall[:, h * HID:(h + 1) * HID],
                            preferred_element_type=jnp.float32).astype(jnp.bfloat16))
    h1 = jnp.concatenate(vals, axis=-1)

    Wf2 = Wf2_ref[...].astype(jnp.bfloat16)
    F2 = (jnp.dot(h1, Wf2, preferred_element_type=jnp.float32)
          + fb2_ref[0:1, :]).astype(jnp.bfloat16)
    G2 = (lax.dot_general(Wf2, h1, (((0,), (1,)), ((), ())),
                          preferred_element_type=jnp.float32)
          + fbc2_ref[...]).astype(jnp.bfloat16)

    S2all = (jnp.dot(h1, W2_ref[...].astype(jnp.bfloat16),
                     preferred_element_type=jnp.float32)
             + s2bias_ref[0:1, :]).astype(jnp.bfloat16)

    acc = jnp.zeros((ROWS, OUT), jnp.float32)
    for h in range(HEADS):
        Z2 = F2[:, 2 * h:2 * h + 1] + G2[2 * h + 1:2 * h + 2, :]
        Z2 = jnp.maximum(Z2, jnp.bfloat16(0.2) * Z2)
        E2 = jnp.exp(Z2) * maskb
        denom2 = jnp.sum(E2.astype(jnp.float32), axis=-1, keepdims=True)
        recip2 = 1.0 / denom2
        V2 = jnp.dot(E2, S2all[:, h * OUT:(h + 1) * OUT],
                     preferred_element_type=jnp.float32) * recip2
        acc = acc + V2
    out = acc.reshape(BB, NB1, OUT)[:, 0, :] * (1.0 / HEADS) + bmean_ref[0:1, :]
    out_ref[...] = out


def _attention(seq, W1all, Wf1, fb1, fbc1, W2all, Wf2, fb2, fbc2, s2bias, bmean):
    grid = BATCH // BB
    full = lambda *s: pl.BlockSpec(s, lambda i: tuple(0 for _ in s))
    return pl.pallas_call(
        _att_body,
        grid=(grid,),
        in_specs=[
            pl.BlockSpec((ROWS, FEAT), lambda i: (i, 0)),
            full(FEAT, HEADS * HID),
            full(FEAT, 2 * HEADS),
            full(1, 2 * HEADS),
            full(2 * HEADS, 1),
            full(HEADS * HID, HEADS * OUT),
            full(HEADS * HID, 2 * HEADS),
            full(1, 2 * HEADS),
            full(2 * HEADS, 1),
            full(1, HEADS * OUT),
            full(1, OUT),
        ],
        out_specs=pl.BlockSpec((BB, OUT), lambda i: (i, 0)),
        out_shape=jax.ShapeDtypeStruct((BATCH, OUT), jnp.float32),
    )(seq, W1all, Wf1, fb1, fbc1, W2all, Wf2, fb2, fbc2, s2bias, bmean)


def kernel(inputs, neighbors, table, W1, f1w1, f1b1, f2w1, f2b1, bout1,
           W2, f1w2, f1b2, f2w2, f2b2, bout2):
    idx = jnp.concatenate([inputs[:, None], neighbors], axis=1)
    idx = idx.reshape(1, BATCH * NB1).astype(jnp.int32)
    seq = _gather(table, idx)
    Wf1 = jnp.stack([jnp.einsum('do,ok->dk', W1[h], w)[:, 0]
                     for h in range(HEADS) for w in (f1w1[h], f2w1[h])], axis=-1)
    Wf2 = jnp.stack([jnp.einsum('do,ok->dk', W2[h], w)[:, 0]
                     for h in range(HEADS) for w in (f1w2[h], f2w2[h])], axis=-1)
    W1all = jnp.concatenate([W1[h] for h in range(HEADS)], axis=-1)
    W2all = jnp.concatenate([W2[h] for h in range(HEADS)], axis=-1)
    boutcat = jnp.concatenate([bout1[h] for h in range(HEADS)])
    b1 = jnp.stack([b for h in range(HEADS) for b in (f1b1[h, 0], f2b1[h, 0])])
    b2 = jnp.stack([b for h in range(HEADS) for b in (f1b2[h, 0], f2b2[h, 0])])
    b2 = b2 + boutcat @ Wf2
    s2bias = (boutcat @ W2all)[None, :]
    bmean = jnp.mean(bout2, axis=0)[None, :]
    fb1, fbc1 = b1[None, :], b1[:, None]
    fb2, fbc2 = b2[None, :], b2[:, None]
    return _attention(seq, W1all, Wf1, fb1, fbc1, W2all, Wf2, fb2, fbc2,
                      s2bias, bmean)


# speedup vs baseline: 1.0415x; 1.0415x over previous
import functools

import jax
import jax.numpy as jnp
from jax import lax
from jax.experimental import pallas as pl
from jax.experimental.pallas import tpu as pltpu
from jax.experimental.pallas import tpu_sc as plsc

BATCH = 2048
NB1 = 32
FEAT = 256
HID = 256
OUT = 128
HEADS = 2
BB = 8
ROWS = BB * NB1
GW = 128


def _leaky(x):
    return jnp.where(x >= 0, x, 0.2 * x)


CHUNKS = 4


def _gather(table, idx):
    n = idx.shape[1]
    mesh = plsc.VectorSubcoreMesh(core_axis_name="core", subcore_axis_name="subcore")

    @functools.partial(
        pl.kernel,
        out_type=jax.ShapeDtypeStruct((n, FEAT), jnp.float32),
        mesh=mesh,
    )
    def gk(table_hbm, idx_hbm, out_hbm):
        def body(i_vmem, o_vmem):
            pltpu.sync_copy(table_hbm.at[i_vmem.at[0]], o_vmem)

        pltpu.emit_pipeline(
            body,
            grid=(n // GW,),
            in_specs=[pl.BlockSpec((1, GW), index_map=lambda i: (0, i))],
            out_specs=[pl.BlockSpec((GW, FEAT), index_map=lambda i: (i, 0))],
            core_axis_name=("core", "subcore"),
            dimension_semantics=(pltpu.PARALLEL,),
        )(idx_hbm, out_hbm)

    return gk(table, idx)


def _att_body(seq_ref, W1_ref, Wf1_ref, fb1_ref, fbc1_ref,
              W2_ref, Wf2_ref, fb2_ref, fbc2_ref, s2bias_ref, bmean_ref, out_ref):
    X = seq_ref[...].astype(jnp.bfloat16)
    rid = lax.broadcasted_iota(jnp.int32, (ROWS, ROWS), 0) // NB1
    cid = lax.broadcasted_iota(jnp.int32, (ROWS, ROWS), 1) // NB1
    maskb = (rid == cid).astype(jnp.bfloat16)

    Wf1 = Wf1_ref[...].astype(jnp.bfloat16)
    F1 = (jnp.dot(X, Wf1, preferred_element_type=jnp.float32)
          + fb1_ref[0:1, :]).astype(jnp.bfloat16)
    G1 = (lax.dot_general(Wf1, X, (((0,), (1,)), ((), ())),
                          preferred_element_type=jnp.float32)
          + fbc1_ref[...]).astype(jnp.bfloat16)

    Sall = jnp.dot(X, W1_ref[...].astype(jnp.bfloat16),
                   preferred_element_type=jnp.float32).astype(jnp.bfloat16)
    vals = []
    for h in range(HEADS):
        Z = F1[:, 2 * h:2 * h + 1] + G1[2 * h + 1:2 * h + 2, :]
        Z = jnp.maximum(Z, jnp.bfloat16(0.2) * Z)
        E = jnp.exp(Z) * maskb
        denom = jnp.sum(E.astype(jnp.float32), axis=-1, keepdims=True)
        A = E * (1.0 / denom).astype(jnp.bfloat16)
        vals.append(jnp.dot(A, Sall[:, h * HID:(h + 1) * HID],
                            preferred_element_type=jnp.float32).astype(jnp.bfloat16))
    h1 = jnp.concatenate(vals, axis=-1)

    Wf2 = Wf2_ref[...].astype(jnp.bfloat16)
    F2 = (jnp.dot(h1, Wf2, preferred_element_type=jnp.float32)
          + fb2_ref[0:1, :]).astype(jnp.bfloat16)
    G2 = (lax.dot_general(Wf2, h1, (((0,), (1,)), ((), ())),
                          preferred_element_type=jnp.float32)
          + fbc2_ref[...]).astype(jnp.bfloat16)

    S2all = (jnp.dot(h1, W2_ref[...].astype(jnp.bfloat16),
                     preferred_element_type=jnp.float32)
             + s2bias_ref[0:1, :]).astype(jnp.bfloat16)

    acc = jnp.zeros((ROWS, OUT), jnp.float32)
    for h in range(HEADS):
        Z2 = F2[:, 2 * h:2 * h + 1] + G2[2 * h + 1:2 * h + 2, :]
        Z2 = jnp.maximum(Z2, jnp.bfloat16(0.2) * Z2)
        E2 = jnp.exp(Z2) * maskb
        denom2 = jnp.sum(E2.astype(jnp.float32), axis=-1, keepdims=True)
        recip2 = 1.0 / denom2
        V2 = jnp.dot(E2, S2all[:, h * OUT:(h + 1) * OUT],
                     preferred_element_type=jnp.float32) * recip2
        acc = acc + V2
    out = acc.reshape(BB, NB1, OUT)[:, 0, :] * (1.0 / HEADS) + bmean_ref[0:1, :]
    out_ref[...] = out


def _attention(seq, W1all, Wf1, fb1, fbc1, W2all, Wf2, fb2, fbc2, s2bias, bmean):
    batch = seq.shape[0] // NB1
    grid = batch // BB
    full = lambda *s: pl.BlockSpec(s, lambda i: tuple(0 for _ in s))
    return pl.pallas_call(
        _att_body,
        grid=(grid,),
        in_specs=[
            pl.BlockSpec((ROWS, FEAT), lambda i: (i, 0)),
            full(FEAT, HEADS * HID),
            full(FEAT, 2 * HEADS),
            full(1, 2 * HEADS),
            full(2 * HEADS, 1),
            full(HEADS * HID, HEADS * OUT),
            full(HEADS * HID, 2 * HEADS),
            full(1, 2 * HEADS),
            full(2 * HEADS, 1),
            full(1, HEADS * OUT),
            full(1, OUT),
        ],
        out_specs=pl.BlockSpec((BB, OUT), lambda i: (i, 0)),
        out_shape=jax.ShapeDtypeStruct((batch, OUT), jnp.float32),
    )(seq, W1all, Wf1, fb1, fbc1, W2all, Wf2, fb2, fbc2, s2bias, bmean)


def kernel(inputs, neighbors, table, W1, f1w1, f1b1, f2w1, f2b1, bout1,
           W2, f1w2, f1b2, f2w2, f2b2, bout2):
    idx = jnp.concatenate([inputs[:, None], neighbors], axis=1)
    idx = idx.reshape(1, BATCH * NB1).astype(jnp.int32)
    Wf1 = jnp.einsum('hdo,hok->dhk',
                     W1, jnp.concatenate([f1w1, f2w1], -1)).reshape(FEAT, -1)
    Wf2 = jnp.einsum('hdo,hok->dhk',
                     W2, jnp.concatenate([f1w2, f2w2], -1)).reshape(HEADS * HID, -1)
    W1all = jnp.moveaxis(W1, 0, 1).reshape(FEAT, -1)
    W2all = jnp.moveaxis(W2, 0, 1).reshape(HEADS * HID, -1)
    boutcat = bout1.reshape(-1)
    b1 = jnp.concatenate([f1b1, f2b1], -1).reshape(-1)
    b2 = jnp.concatenate([f1b2, f2b2], -1).reshape(-1)
    b2 = b2 + boutcat @ Wf2
    s2bias = (boutcat @ W2all)[None, :]
    bmean = jnp.mean(bout2, axis=0)[None, :]
    fb1, fbc1 = b1[None, :], b1[:, None]
    fb2, fbc2 = b2[None, :], b2[:, None]
    outs = []
    rows_per_chunk = (BATCH // CHUNKS) * NB1
    for c in range(CHUNKS):
        idxc = idx[:, c * rows_per_chunk:(c + 1) * rows_per_chunk]
        seqc = _gather(table, idxc)
        outs.append(_attention(seqc, W1all, Wf1, fb1, fbc1, W2all, Wf2,
                               fb2, fbc2, s2bias, bmean))
    return jnp.concatenate(outs, axis=0)


# speedup vs baseline: 1.1737x; 1.1269x over previous
import functools

import jax
import jax.numpy as jnp
from jax import lax
from jax.experimental import pallas as pl
from jax.experimental.pallas import tpu as pltpu
from jax.experimental.pallas import tpu_sc as plsc

BATCH = 2048
NB1 = 32
FEAT = 256
HID = 256
OUT = 128
HEADS = 2
BB = 8
ROWS = BB * NB1
GPB = 2
GW = 128


def _leaky(x):
    return jnp.where(x >= 0, x, 0.2 * x)


CHUNKS = 4


def _gather(table, idx):
    n = idx.shape[1]
    mesh = plsc.VectorSubcoreMesh(core_axis_name="core", subcore_axis_name="subcore")

    @functools.partial(
        pl.kernel,
        out_type=jax.ShapeDtypeStruct((n, FEAT), jnp.float32),
        mesh=mesh,
    )
    def gk(table_hbm, idx_hbm, out_hbm):
        def body(i_vmem, o_vmem):
            pltpu.sync_copy(table_hbm.at[i_vmem.at[0]], o_vmem)

        pltpu.emit_pipeline(
            body,
            grid=(n // GW,),
            in_specs=[pl.BlockSpec((1, GW), index_map=lambda i: (0, i))],
            out_specs=[pl.BlockSpec((GW, FEAT), index_map=lambda i: (i, 0))],
            core_axis_name=("core", "subcore"),
            dimension_semantics=(pltpu.PARALLEL,),
        )(idx_hbm, out_hbm)

    return gk(table, idx)


def _att_body(seq_ref, W1_ref, Wf1_ref, fb1_ref, fbc1_ref,
              W2_ref, Wf2_ref, fb2_ref, fbc2_ref, s2bias_ref, bmean_ref, out_ref):
    rid = lax.broadcasted_iota(jnp.int32, (ROWS, ROWS), 0) // NB1
    cid = lax.broadcasted_iota(jnp.int32, (ROWS, ROWS), 1) // NB1
    maskb = (rid == cid).astype(jnp.bfloat16)
    for g in range(GPB):
        _att_group(seq_ref, W1_ref, Wf1_ref, fb1_ref, fbc1_ref, W2_ref,
                   Wf2_ref, fb2_ref, fbc2_ref, s2bias_ref, bmean_ref,
                   out_ref, maskb, g)


def _att_group(seq_ref, W1_ref, Wf1_ref, fb1_ref, fbc1_ref,
               W2_ref, Wf2_ref, fb2_ref, fbc2_ref, s2bias_ref, bmean_ref,
               out_ref, maskb, g):
    X = seq_ref[g * ROWS:(g + 1) * ROWS, :].astype(jnp.bfloat16)

    Wf1 = Wf1_ref[...].astype(jnp.bfloat16)
    F1 = (jnp.dot(X, Wf1, preferred_element_type=jnp.float32)
          + fb1_ref[0:1, :]).astype(jnp.bfloat16)
    G1 = (lax.dot_general(Wf1, X, (((0,), (1,)), ((), ())),
                          preferred_element_type=jnp.float32)
          + fbc1_ref[...]).astype(jnp.bfloat16)

    Sall = jnp.dot(X, W1_ref[...].astype(jnp.bfloat16),
                   preferred_element_type=jnp.float32).astype(jnp.bfloat16)
    vals = []
    for h in range(HEADS):
        Z = F1[:, 2 * h:2 * h + 1] + G1[2 * h + 1:2 * h + 2, :]
        Z = jnp.maximum(Z, jnp.bfloat16(0.2) * Z)
        E = jnp.exp(Z) * maskb
        denom = jnp.sum(E.astype(jnp.float32), axis=-1, keepdims=True)
        A = E * (1.0 / denom).astype(jnp.bfloat16)
        vals.append(jnp.dot(A, Sall[:, h * HID:(h + 1) * HID],
                            preferred_element_type=jnp.float32).astype(jnp.bfloat16))
    h1 = jnp.concatenate(vals, axis=-1)

    Wf2 = Wf2_ref[...].astype(jnp.bfloat16)
    F2 = (jnp.dot(h1, Wf2, preferred_element_type=jnp.float32)
          + fb2_ref[0:1, :]).astype(jnp.bfloat16)
    G2 = (lax.dot_general(Wf2, h1, (((0,), (1,)), ((), ())),
                          preferred_element_type=jnp.float32)
          + fbc2_ref[...]).astype(jnp.bfloat16)

    S2all = (jnp.dot(h1, W2_ref[...].astype(jnp.bfloat16),
                     preferred_element_type=jnp.float32)
             + s2bias_ref[0:1, :]).astype(jnp.bfloat16)

    acc = jnp.zeros((ROWS, OUT), jnp.float32)
    for h in range(HEADS):
        Z2 = F2[:, 2 * h:2 * h + 1] + G2[2 * h + 1:2 * h + 2, :]
        Z2 = jnp.maximum(Z2, jnp.bfloat16(0.2) * Z2)
        E2 = jnp.exp(Z2) * maskb
        denom2 = jnp.sum(E2.astype(jnp.float32), axis=-1, keepdims=True)
        recip2 = 1.0 / denom2
        V2 = jnp.dot(E2, S2all[:, h * OUT:(h + 1) * OUT],
                     preferred_element_type=jnp.float32) * recip2
        acc = acc + V2
    out = acc.reshape(BB, NB1, OUT)[:, 0, :] * (1.0 / HEADS) + bmean_ref[0:1, :]
    out_ref[g * BB:(g + 1) * BB, :] = out


def _attention(seq, W1all, Wf1, fb1, fbc1, W2all, Wf2, fb2, fbc2, s2bias, bmean):
    batch = seq.shape[0] // NB1
    grid = batch // (BB * GPB)
    full = lambda *s: pl.BlockSpec(s, lambda i: tuple(0 for _ in s))
    return pl.pallas_call(
        _att_body,
        grid=(grid,),
        in_specs=[
            pl.BlockSpec((GPB * ROWS, FEAT), lambda i: (i, 0)),
            full(FEAT, HEADS * HID),
            full(FEAT, 2 * HEADS),
            full(1, 2 * HEADS),
            full(2 * HEADS, 1),
            full(HEADS * HID, HEADS * OUT),
            full(HEADS * HID, 2 * HEADS),
            full(1, 2 * HEADS),
            full(2 * HEADS, 1),
            full(1, HEADS * OUT),
            full(1, OUT),
        ],
        out_specs=pl.BlockSpec((GPB * BB, OUT), lambda i: (i, 0)),
        out_shape=jax.ShapeDtypeStruct((batch, OUT), jnp.float32),
    )(seq, W1all, Wf1, fb1, fbc1, W2all, Wf2, fb2, fbc2, s2bias, bmean)


def kernel(inputs, neighbors, table, W1, f1w1, f1b1, f2w1, f2b1, bout1,
           W2, f1w2, f1b2, f2w2, f2b2, bout2):
    idx = jnp.concatenate([inputs[:, None], neighbors], axis=1)
    idx = idx.reshape(1, BATCH * NB1).astype(jnp.int32)
    Wf1 = jnp.einsum('hdo,hok->dhk',
                     W1, jnp.concatenate([f1w1, f2w1], -1)).reshape(FEAT, -1)
    Wf2 = jnp.einsum('hdo,hok->dhk',
                     W2, jnp.concatenate([f1w2, f2w2], -1)).reshape(HEADS * HID, -1)
    W1all = jnp.moveaxis(W1, 0, 1).reshape(FEAT, -1)
    W2all = jnp.moveaxis(W2, 0, 1).reshape(HEADS * HID, -1)
    boutcat = bout1.reshape(-1)
    b1 = jnp.concatenate([f1b1, f2b1], -1).reshape(-1)
    b2 = jnp.concatenate([f1b2, f2b2], -1).reshape(-1)
    b2 = b2 + boutcat @ Wf2
    s2bias = (boutcat @ W2all)[None, :]
    bmean = jnp.mean(bout2, axis=0)[None, :]
    fb1, fbc1 = b1[None, :], b1[:, None]
    fb2, fbc2 = b2[None, :], b2[:, None]
    outs = []
    rows_per_chunk = (BATCH // CHUNKS) * NB1
    for c in range(CHUNKS):
        idxc = idx[:, c * rows_per_chunk:(c + 1) * rows_per_chunk]
        seqc = _gather(table, idxc)
        outs.append(_attention(seqc, W1all, Wf1, fb1, fbc1, W2all, Wf2,
                               fb2, fbc2, s2bias, bmean))
    return jnp.concatenate(outs, axis=0)


# speedup vs baseline: 1.2664x; 1.0790x over previous
import functools

import jax
import jax.numpy as jnp
from jax import lax
from jax.experimental import pallas as pl
from jax.experimental.pallas import tpu as pltpu
from jax.experimental.pallas import tpu_sc as plsc

BATCH = 2048
NB1 = 32
FEAT = 256
HID = 256
OUT = 128
HEADS = 2
BB = 8
ROWS = BB * NB1
GPB = 4
GW = 128


def _leaky(x):
    return jnp.where(x >= 0, x, 0.2 * x)


CHUNKS = 4


def _gather(table, idx):
    n = idx.shape[1]
    mesh = plsc.VectorSubcoreMesh(core_axis_name="core", subcore_axis_name="subcore")

    @functools.partial(
        pl.kernel,
        out_type=jax.ShapeDtypeStruct((n, FEAT), jnp.float32),
        mesh=mesh,
    )
    def gk(table_hbm, idx_hbm, out_hbm):
        def body(i_vmem, o_vmem):
            pltpu.sync_copy(table_hbm.at[i_vmem.at[0]], o_vmem)

        pltpu.emit_pipeline(
            body,
            grid=(n // GW,),
            in_specs=[pl.BlockSpec((1, GW), index_map=lambda i: (0, i))],
            out_specs=[pl.BlockSpec((GW, FEAT), index_map=lambda i: (i, 0))],
            core_axis_name=("core", "subcore"),
            dimension_semantics=(pltpu.PARALLEL,),
        )(idx_hbm, out_hbm)

    return gk(table, idx)


def _att_body(seq_ref, W1_ref, Wf1_ref, fb1_ref, fbc1_ref,
              W2_ref, Wf2_ref, fb2_ref, fbc2_ref, s2bias_ref, bmean_ref, out_ref):
    rid = lax.broadcasted_iota(jnp.int32, (ROWS, ROWS), 0) // NB1
    cid = lax.broadcasted_iota(jnp.int32, (ROWS, ROWS), 1) // NB1
    maskb = (rid == cid).astype(jnp.bfloat16)
    for g in range(GPB):
        _att_group(seq_ref, W1_ref, Wf1_ref, fb1_ref, fbc1_ref, W2_ref,
                   Wf2_ref, fb2_ref, fbc2_ref, s2bias_ref, bmean_ref,
                   out_ref, maskb, g)


def _att_group(seq_ref, W1_ref, Wf1_ref, fb1_ref, fbc1_ref,
               W2_ref, Wf2_ref, fb2_ref, fbc2_ref, s2bias_ref, bmean_ref,
               out_ref, maskb, g):
    X = seq_ref[g * ROWS:(g + 1) * ROWS, :].astype(jnp.bfloat16)

    Wf1 = Wf1_ref[...].astype(jnp.bfloat16)
    F1 = (jnp.dot(X, Wf1, preferred_element_type=jnp.float32)
          + fb1_ref[0:1, :]).astype(jnp.bfloat16)
    G1 = (lax.dot_general(Wf1, X, (((0,), (1,)), ((), ())),
                          preferred_element_type=jnp.float32)
          + fbc1_ref[...]).astype(jnp.bfloat16)

    Sall = jnp.dot(X, W1_ref[...].astype(jnp.bfloat16),
                   preferred_element_type=jnp.float32).astype(jnp.bfloat16)
    vals = []
    for h in range(HEADS):
        Z = F1[:, 2 * h:2 * h + 1] + G1[2 * h + 1:2 * h + 2, :]
        Z = jnp.maximum(Z, jnp.bfloat16(0.2) * Z)
        E = jnp.exp(Z) * maskb
        denom = jnp.sum(E.astype(jnp.float32), axis=-1, keepdims=True)
        A = E * (1.0 / denom).astype(jnp.bfloat16)
        vals.append(jnp.dot(A, Sall[:, h * HID:(h + 1) * HID],
                            preferred_element_type=jnp.float32).astype(jnp.bfloat16))
    h1 = jnp.concatenate(vals, axis=-1)

    Wf2 = Wf2_ref[...].astype(jnp.bfloat16)
    F2 = (jnp.dot(h1, Wf2, preferred_element_type=jnp.float32)
          + fb2_ref[0:1, :]).astype(jnp.bfloat16)
    G2 = (lax.dot_general(Wf2, h1, (((0,), (1,)), ((), ())),
                          preferred_element_type=jnp.float32)
          + fbc2_ref[...]).astype(jnp.bfloat16)

    S2all = (jnp.dot(h1, W2_ref[...].astype(jnp.bfloat16),
                     preferred_element_type=jnp.float32)
             + s2bias_ref[0:1, :]).astype(jnp.bfloat16)

    acc = jnp.zeros((ROWS, OUT), jnp.float32)
    for h in range(HEADS):
        Z2 = F2[:, 2 * h:2 * h + 1] + G2[2 * h + 1:2 * h + 2, :]
        Z2 = jnp.maximum(Z2, jnp.bfloat16(0.2) * Z2)
        E2 = jnp.exp(Z2) * maskb
        denom2 = jnp.sum(E2.astype(jnp.float32), axis=-1, keepdims=True)
        recip2 = 1.0 / denom2
        V2 = jnp.dot(E2, S2all[:, h * OUT:(h + 1) * OUT],
                     preferred_element_type=jnp.float32) * recip2
        acc = acc + V2
    out = acc.reshape(BB, NB1, OUT)[:, 0, :] * (1.0 / HEADS) + bmean_ref[0:1, :]
    out_ref[g * BB:(g + 1) * BB, :] = out


def _attention(seq, W1all, Wf1, fb1, fbc1, W2all, Wf2, fb2, fbc2, s2bias, bmean):
    batch = seq.shape[0] // NB1
    grid = batch // (BB * GPB)
    full = lambda *s: pl.BlockSpec(s, lambda i: tuple(0 for _ in s))
    return pl.pallas_call(
        _att_body,
        grid=(grid,),
        in_specs=[
            pl.BlockSpec((GPB * ROWS, FEAT), lambda i: (i, 0)),
            full(FEAT, HEADS * HID),
            full(FEAT, 2 * HEADS),
            full(1, 2 * HEADS),
            full(2 * HEADS, 1),
            full(HEADS * HID, HEADS * OUT),
            full(HEADS * HID, 2 * HEADS),
            full(1, 2 * HEADS),
            full(2 * HEADS, 1),
            full(1, HEADS * OUT),
            full(1, OUT),
        ],
        out_specs=pl.BlockSpec((GPB * BB, OUT), lambda i: (i, 0)),
        out_shape=jax.ShapeDtypeStruct((batch, OUT), jnp.float32),
    )(seq, W1all, Wf1, fb1, fbc1, W2all, Wf2, fb2, fbc2, s2bias, bmean)


def kernel(inputs, neighbors, table, W1, f1w1, f1b1, f2w1, f2b1, bout1,
           W2, f1w2, f1b2, f2w2, f2b2, bout2):
    idx = jnp.concatenate([inputs[:, None], neighbors], axis=1)
    idx = idx.reshape(1, BATCH * NB1).astype(jnp.int32)
    Wf1 = jnp.einsum('hdo,hok->dhk',
                     W1, jnp.concatenate([f1w1, f2w1], -1)).reshape(FEAT, -1)
    Wf2 = jnp.einsum('hdo,hok->dhk',
                     W2, jnp.concatenate([f1w2, f2w2], -1)).reshape(HEADS * HID, -1)
    W1all = jnp.moveaxis(W1, 0, 1).reshape(FEAT, -1)
    W2all = jnp.moveaxis(W2, 0, 1).reshape(HEADS * HID, -1)
    boutcat = bout1.reshape(-1)
    b1 = jnp.concatenate([f1b1, f2b1], -1).reshape(-1)
    b2 = jnp.concatenate([f1b2, f2b2], -1).reshape(-1)
    b2 = b2 + boutcat @ Wf2
    s2bias = (boutcat @ W2all)[None, :]
    bmean = jnp.mean(bout2, axis=0)[None, :]
    fb1, fbc1 = b1[None, :], b1[:, None]
    fb2, fbc2 = b2[None, :], b2[:, None]
    outs = []
    rows_per_chunk = (BATCH // CHUNKS) * NB1
    for c in range(CHUNKS):
        idxc = idx[:, c * rows_per_chunk:(c + 1) * rows_per_chunk]
        seqc = _gather(table, idxc)
        outs.append(_attention(seqc, W1all, Wf1, fb1, fbc1, W2all, Wf2,
                               fb2, fbc2, s2bias, bmean))
    return jnp.concatenate(outs, axis=0)


# speedup vs baseline: 1.3093x; 1.0339x over previous
import functools

import jax
import jax.numpy as jnp
from jax import lax
from jax.experimental import pallas as pl
from jax.experimental.pallas import tpu as pltpu
from jax.experimental.pallas import tpu_sc as plsc

BATCH = 2048
NB1 = 32
FEAT = 256
HID = 256
OUT = 128
HEADS = 2
BB = 8
ROWS = BB * NB1
GPB = 8
GW = 128


def _leaky(x):
    return jnp.where(x >= 0, x, 0.2 * x)


CHUNKS = 4


def _gather(table, idx):
    n = idx.shape[1]
    mesh = plsc.VectorSubcoreMesh(core_axis_name="core", subcore_axis_name="subcore")

    @functools.partial(
        pl.kernel,
        out_type=jax.ShapeDtypeStruct((n, FEAT), jnp.float32),
        mesh=mesh,
    )
    def gk(table_hbm, idx_hbm, out_hbm):
        def body(i_vmem, o_vmem):
            pltpu.sync_copy(table_hbm.at[i_vmem.at[0]], o_vmem)

        pltpu.emit_pipeline(
            body,
            grid=(n // GW,),
            in_specs=[pl.BlockSpec((1, GW), index_map=lambda i: (0, i))],
            out_specs=[pl.BlockSpec((GW, FEAT), index_map=lambda i: (i, 0))],
            core_axis_name=("core", "subcore"),
            dimension_semantics=(pltpu.PARALLEL,),
        )(idx_hbm, out_hbm)

    return gk(table, idx)


def _att_body(seq_ref, W1_ref, Wf1_ref, fb1_ref, fbc1_ref,
              W2_ref, Wf2_ref, fb2_ref, fbc2_ref, s2bias_ref, bmean_ref, out_ref):
    rid = lax.broadcasted_iota(jnp.int32, (ROWS, ROWS), 0) // NB1
    cid = lax.broadcasted_iota(jnp.int32, (ROWS, ROWS), 1) // NB1
    maskb = (rid == cid).astype(jnp.bfloat16)
    for g in range(GPB):
        _att_group(seq_ref, W1_ref, Wf1_ref, fb1_ref, fbc1_ref, W2_ref,
                   Wf2_ref, fb2_ref, fbc2_ref, s2bias_ref, bmean_ref,
                   out_ref, maskb, g)


def _att_group(seq_ref, W1_ref, Wf1_ref, fb1_ref, fbc1_ref,
               W2_ref, Wf2_ref, fb2_ref, fbc2_ref, s2bias_ref, bmean_ref,
               out_ref, maskb, g):
    X = seq_ref[g * ROWS:(g + 1) * ROWS, :].astype(jnp.bfloat16)

    Wf1 = Wf1_ref[...].astype(jnp.bfloat16)
    F1 = (jnp.dot(X, Wf1, preferred_element_type=jnp.float32)
          + fb1_ref[0:1, :]).astype(jnp.bfloat16)
    G1 = (lax.dot_general(Wf1, X, (((0,), (1,)), ((), ())),
                          preferred_element_type=jnp.float32)
          + fbc1_ref[...]).astype(jnp.bfloat16)

    Sall = jnp.dot(X, W1_ref[...].astype(jnp.bfloat16),
                   preferred_element_type=jnp.float32).astype(jnp.bfloat16)
    vals = []
    for h in range(HEADS):
        Z = F1[:, 2 * h:2 * h + 1] + G1[2 * h + 1:2 * h + 2, :]
        Z = jnp.maximum(Z, jnp.bfloat16(0.2) * Z)
        E = jnp.exp(Z) * maskb
        denom = jnp.sum(E.astype(jnp.float32), axis=-1, keepdims=True)
        A = E * (1.0 / denom).astype(jnp.bfloat16)
        vals.append(jnp.dot(A, Sall[:, h * HID:(h + 1) * HID],
                            preferred_element_type=jnp.float32).astype(jnp.bfloat16))
    h1 = jnp.concatenate(vals, axis=-1)

    Wf2 = Wf2_ref[...].astype(jnp.bfloat16)
    F2 = (jnp.dot(h1, Wf2, preferred_element_type=jnp.float32)
          + fb2_ref[0:1, :]).astype(jnp.bfloat16)
    G2 = (lax.dot_general(Wf2, h1, (((0,), (1,)), ((), ())),
                          preferred_element_type=jnp.float32)
          + fbc2_ref[...]).astype(jnp.bfloat16)

    S2all = (jnp.dot(h1, W2_ref[...].astype(jnp.bfloat16),
                     preferred_element_type=jnp.float32)
             + s2bias_ref[0:1, :]).astype(jnp.bfloat16)

    acc = jnp.zeros((ROWS, OUT), jnp.float32)
    for h in range(HEADS):
        Z2 = F2[:, 2 * h:2 * h + 1] + G2[2 * h + 1:2 * h + 2, :]
        Z2 = jnp.maximum(Z2, jnp.bfloat16(0.2) * Z2)
        E2 = jnp.exp(Z2) * maskb
        denom2 = jnp.sum(E2.astype(jnp.float32), axis=-1, keepdims=True)
        recip2 = 1.0 / denom2
        V2 = jnp.dot(E2, S2all[:, h * OUT:(h + 1) * OUT],
                     preferred_element_type=jnp.float32) * recip2
        acc = acc + V2
    out = acc.reshape(BB, NB1, OUT)[:, 0, :] * (1.0 / HEADS) + bmean_ref[0:1, :]
    out_ref[g * BB:(g + 1) * BB, :] = out


def _attention(seq, W1all, Wf1, fb1, fbc1, W2all, Wf2, fb2, fbc2, s2bias, bmean):
    batch = seq.shape[0] // NB1
    grid = batch // (BB * GPB)
    full = lambda *s: pl.BlockSpec(s, lambda i: tuple(0 for _ in s))
    return pl.pallas_call(
        _att_body,
        grid=(grid,),
        in_specs=[
            pl.BlockSpec((GPB * ROWS, FEAT), lambda i: (i, 0)),
            full(FEAT, HEADS * HID),
            full(FEAT, 2 * HEADS),
            full(1, 2 * HEADS),
            full(2 * HEADS, 1),
            full(HEADS * HID, HEADS * OUT),
            full(HEADS * HID, 2 * HEADS),
            full(1, 2 * HEADS),
            full(2 * HEADS, 1),
            full(1, HEADS * OUT),
            full(1, OUT),
        ],
        out_specs=pl.BlockSpec((GPB * BB, OUT), lambda i: (i, 0)),
        out_shape=jax.ShapeDtypeStruct((batch, OUT), jnp.float32),
    )(seq, W1all, Wf1, fb1, fbc1, W2all, Wf2, fb2, fbc2, s2bias, bmean)


def kernel(inputs, neighbors, table, W1, f1w1, f1b1, f2w1, f2b1, bout1,
           W2, f1w2, f1b2, f2w2, f2b2, bout2):
    idx = jnp.concatenate([inputs[:, None], neighbors], axis=1)
    idx = idx.reshape(1, BATCH * NB1).astype(jnp.int32)
    Wf1 = jnp.einsum('hdo,hok->dhk',
                     W1, jnp.concatenate([f1w1, f2w1], -1)).reshape(FEAT, -1)
    Wf2 = jnp.einsum('hdo,hok->dhk',
                     W2, jnp.concatenate([f1w2, f2w2], -1)).reshape(HEADS * HID, -1)
    W1all = jnp.moveaxis(W1, 0, 1).reshape(FEAT, -1)
    W2all = jnp.moveaxis(W2, 0, 1).reshape(HEADS * HID, -1)
    boutcat = bout1.reshape(-1)
    b1 = jnp.concatenate([f1b1, f2b1], -1).reshape(-1)
    b2 = jnp.concatenate([f1b2, f2b2], -1).reshape(-1)
    b2 = b2 + boutcat @ Wf2
    s2bias = (boutcat @ W2all)[None, :]
    bmean = jnp.mean(bout2, axis=0)[None, :]
    fb1, fbc1 = b1[None, :], b1[:, None]
    fb2, fbc2 = b2[None, :], b2[:, None]
    outs = []
    rows_per_chunk = (BATCH // CHUNKS) * NB1
    for c in range(CHUNKS):
        idxc = idx[:, c * rows_per_chunk:(c + 1) * rows_per_chunk]
        seqc = _gather(table, idxc)
        outs.append(_attention(seqc, W1all, Wf1, fb1, fbc1, W2all, Wf2,
                               fb2, fbc2, s2bias, bmean))
    return jnp.concatenate(outs, axis=0)
